# Initial kernel scaffold; baseline (speedup 1.0000x reference)
import jax
import jax.numpy as jnp
from jax.experimental import pallas as pl


def kernel(encoded, embeddings):
    raise NotImplementedError("write your pallas kernel here")


# speedup vs baseline: 6.7540x; 6.7540x over previous
import functools

import jax
import jax.numpy as jnp
from jax import lax
from jax.experimental import pallas as pl
from jax.experimental.pallas import tpu as pltpu

_B = 4096
_L = 64
_K = 512
_BB = 128


def _vq_body(x_ref, e_ref, out_ref, loss_ref):
    pid = pl.program_id(0)
    x = x_ref[...]

    def step(k, carry):
        best, val = carry
        e = e_ref[pl.ds(k, 1), :]
        score = (x - 0.5 * e) * e
        better = score > best
        best = jnp.where(better, score, best)
        val = jnp.where(better, jnp.broadcast_to(e, x.shape), val)
        return best, val

    init = (jnp.full(x.shape, -jnp.inf, jnp.float32),
            jnp.zeros(x.shape, jnp.float32))
    _, val = lax.fori_loop(0, _K, step, init)
    out_ref[...] = val
    d = val - x
    part = 2.0 * jnp.sum(d * d)

    @pl.when(pid == 0)
    def _():
        loss_ref[0, 0] = 0.0

    loss_ref[0, 0] += part


@jax.jit
def _vq_tc(encoded, emb_t):
    grid = (_B // _BB,)
    latent, loss = pl.pallas_call(
        _vq_body,
        grid=grid,
        in_specs=[
            pl.BlockSpec((_BB, _L), lambda i: (i, 0)),
            pl.BlockSpec((_K, _L), lambda i: (0, 0)),
        ],
        out_specs=[
            pl.BlockSpec((_BB, _L), lambda i: (i, 0)),
            pl.BlockSpec(memory_space=pltpu.SMEM),
        ],
        out_shape=[
            jax.ShapeDtypeStruct((_B, _L), jnp.float32),
            jax.ShapeDtypeStruct((1, 1), jnp.float32),
        ],
    )(encoded, emb_t)
    return latent, loss[0, 0]


def kernel(encoded, embeddings):
    emb_t = embeddings[0].T
    return _vq_tc(encoded, emb_t)
